# baseline (speedup 1.0000x reference)
import functools

import jax
import jax.numpy as jnp
from jax import lax
from jax.experimental import pallas as pl
from jax.experimental.pallas import tpu as pltpu
from jax.experimental.pallas import tpu_sc as plsc

B, S, D = 4, 8192, 1024

NW = 32
ROWS_W = S // NW
R = 8
N_CHUNKS = ROWS_W // R
XPH = 3

_sc_mesh = plsc.VectorSubcoreMesh(core_axis_name="c", subcore_axis_name="s")


@functools.partial(
    pl.kernel,
    mesh=_sc_mesh,
    out_type=jax.ShapeDtypeStruct((B, S, D), jnp.float32),
    scratch_types=[
        pltpu.VMEM((2, R, D), jnp.float32),
        pltpu.VMEM((XPH, 4, R, D), jnp.float32),
        pltpu.SemaphoreType.DMA((2,)),
        pltpu.SemaphoreType.DMA((XPH, 4)),
        pltpu.SemaphoreType.DMA((XPH, 4)),
    ],
)
def _sc_add(x_hbm, pe_hbm, out_hbm, pe_buf, x_buf, pe_sem, x_sem, out_sem):
    c_ax = lax.axis_index("c")
    s_ax = lax.axis_index("s")
    w = s_ax * 2 + c_ax
    row0 = w * ROWS_W

    def issue(p2, p3, ci):
        r = row0 + ci * R
        pltpu.async_copy(pe_hbm.at[pl.ds(r, R), :], pe_buf.at[p2], pe_sem.at[p2])
        for b in range(4):
            pltpu.async_copy(
                x_hbm.at[b, pl.ds(r, R), :], x_buf.at[p3, b], x_sem.at[p3, b]
            )

    issue(0, 0, 0)

    def process(p2, p3, ci):
        n3 = (p3 + 1) % XPH

        @pl.when(ci >= 2)
        def _():
            for b in range(4):
                pltpu.make_async_copy(
                    x_buf.at[n3, b],
                    out_hbm.at[b, pl.ds(row0, R), :],
                    out_sem.at[n3, b],
                ).wait()

        @pl.when(ci < N_CHUNKS - 1)
        def _():
            issue(1 - p2, n3, ci + 1)

        pltpu.make_async_copy(
            pe_hbm.at[pl.ds(row0, R), :], pe_buf.at[p2], pe_sem.at[p2]
        ).wait()
        for b in range(4):
            pltpu.make_async_copy(
                x_hbm.at[b, pl.ds(row0, R), :], x_buf.at[p3, b], x_sem.at[p3, b]
            ).wait()

        def row_body(r, carry):
            for h in range(2):
                hb = h * 512
                pe_vals = [
                    pe_buf[p2, r, pl.ds(hb + k * 16, 16)] for k in range(32)
                ]
                for b in range(4):
                    for k in range(32):
                        sl = pl.ds(hb + k * 16, 16)
                        x_buf[p3, b, r, sl] = x_buf[p3, b, r, sl] + pe_vals[k]
            return carry

        lax.fori_loop(0, R, row_body, 0)

        r = row0 + ci * R
        for b in range(4):
            pltpu.async_copy(
                x_buf.at[p3, b],
                out_hbm.at[b, pl.ds(r, R), :],
                out_sem.at[p3, b],
            )

    def outer(c6, carry):
        for t in range(6):
            process(t % 2, t % 3, c6 * 6 + t)
        return carry

    lax.fori_loop(0, N_CHUNKS // 6, outer, 0)
    process(0, 0, N_CHUNKS - 2)
    process(1, 1, N_CHUNKS - 1)

    for p3, ci in ((0, N_CHUNKS - 2), (1, N_CHUNKS - 1)):
        for b in range(4):
            pltpu.make_async_copy(
                x_buf.at[p3, b], out_hbm.at[b, pl.ds(row0, R), :], out_sem.at[p3, b]
            ).wait()


def kernel(x, pe):
    return _sc_add(x, pe)


# speedup vs baseline: 1.0490x; 1.0490x over previous
import functools

import jax
import jax.numpy as jnp
from jax import lax
from jax.experimental import pallas as pl
from jax.experimental.pallas import tpu as pltpu
from jax.experimental.pallas import tpu_sc as plsc

B, S, D = 4, 8192, 1024

NW = 32
ROWS_W = S // NW
R = 8
N_CHUNKS = ROWS_W // R

_sc_mesh = plsc.VectorSubcoreMesh(core_axis_name="c", subcore_axis_name="s")


@functools.partial(
    pl.kernel,
    mesh=_sc_mesh,
    out_type=jax.ShapeDtypeStruct((B, S, D), jnp.float32),
    scratch_types=[
        pltpu.VMEM((2, R, D), jnp.float32),
        pltpu.VMEM((2, 4, R, D), jnp.float32),
        pltpu.SemaphoreType.DMA((2,)),
        pltpu.SemaphoreType.DMA((2, 4)),
        pltpu.SemaphoreType.DMA((2, 4)),
    ],
)
def _sc_add(x_hbm, pe_hbm, out_hbm, pe_buf, x_buf, pe_sem, x_sem, out_sem):
    c_ax = lax.axis_index("c")
    s_ax = lax.axis_index("s")
    w = s_ax * 2 + c_ax
    row0 = w * ROWS_W

    def issue(ph, ci):
        r = row0 + ci * R
        pltpu.async_copy(pe_hbm.at[pl.ds(r, R), :], pe_buf.at[ph], pe_sem.at[ph])
        for b in range(4):
            pltpu.async_copy(
                x_hbm.at[b, pl.ds(r, R), :], x_buf.at[ph, b], x_sem.at[ph, b]
            )

    issue(0, 0)

    def process(ph, ci):
        @pl.when(ci > 0)
        def _():
            for b in range(4):
                pltpu.make_async_copy(
                    x_buf.at[1 - ph, b],
                    out_hbm.at[b, pl.ds(row0, R), :],
                    out_sem.at[1 - ph, b],
                ).wait()

        @pl.when(ci < N_CHUNKS - 1)
        def _():
            issue(1 - ph, ci + 1)

        pltpu.make_async_copy(
            pe_hbm.at[pl.ds(row0, R), :], pe_buf.at[ph], pe_sem.at[ph]
        ).wait()
        for b in range(4):
            pltpu.make_async_copy(
                x_hbm.at[b, pl.ds(row0, R), :], x_buf.at[ph, b], x_sem.at[ph, b]
            ).wait()

        def row_body(r, carry):
            for h in range(2):
                hb = h * 512
                pe_vals = [
                    pe_buf[ph, r, pl.ds(hb + k * 16, 16)] for k in range(32)
                ]
                for b in range(4):
                    for k in range(32):
                        sl = pl.ds(hb + k * 16, 16)
                        x_buf[ph, b, r, sl] = x_buf[ph, b, r, sl] + pe_vals[k]
            return carry

        lax.fori_loop(0, R, row_body, 0)

        r = row0 + ci * R
        for b in range(4):
            pltpu.async_copy(
                x_buf.at[ph, b],
                out_hbm.at[b, pl.ds(r, R), :],
                out_sem.at[ph, b],
            )

    def outer(c2, carry):
        for ph in range(2):
            process(ph, c2 * 2 + ph)
        return carry

    lax.fori_loop(0, N_CHUNKS // 2, outer, 0)

    for b in range(4):
        pltpu.make_async_copy(
            x_buf.at[1, b], out_hbm.at[b, pl.ds(row0, R), :], out_sem.at[1, b]
        ).wait()


def kernel(x, pe):
    return _sc_add(x, pe)
